# Initial kernel scaffold; baseline (speedup 1.0000x reference)
import jax
import jax.numpy as jnp
from jax.experimental import pallas as pl


def kernel(x, edge_index, batch, Ws, bs, gammas, betas, W_att):
    raise NotImplementedError("write your pallas kernel here")


# speedup vs baseline: 10.3770x; 10.3770x over previous
import functools

import jax
import jax.numpy as jnp
from jax import lax
from jax.experimental import pallas as pl
from jax.experimental.pallas import tpu as pltpu
from jax.experimental.pallas import tpu_sc as plsc

_N = 10000
_E = 320000
_D = 128
_L = 10
_G = 16

_NC = 2
_NS = 16
_NW = _NC * _NS
_K = 80
_EPW = _E // _NW
_NCH = _EPW // _K
_NSEG = 5
_CPS = _NCH // _NSEG
_PAIRS = (_CPS - 1) // 2
_RB = 632
_RL = _N - (_NS - 1) * _RB
_CW = _D

def _zero_slice(z_hbm, acc, s):
    @pl.when(s < _NS - 1)
    def _():
        pltpu.sync_copy(z_hbm, acc.at[pl.ds(s * _RB, _RB)])

    @pl.when(s == _NS - 1)
    def _():
        pltpu.sync_copy(z_hbm.at[pl.ds(0, _RL)],
                        acc.at[pl.ds((_NS - 1) * _RB, _RL)])


def _out_slice(acc, out_hbm, c, s):
    @pl.when(s < _NS - 1)
    def _():
        pltpu.sync_copy(acc.at[pl.ds(s * _RB, _RB)],
                        out_hbm.at[c, pl.ds(s * _RB, _RB)])

    @pl.when(s == _NS - 1)
    def _():
        pltpu.sync_copy(acc.at[pl.ds((_NS - 1) * _RB, _RL)],
                        out_hbm.at[c, pl.ds((_NS - 1) * _RB, _RL)])


def _sc_agg_body(u_hbm, src_hbm, dst_hbm, z_hbm, out_hbm,
                 src_v, dst_v, buf0, buf1, acc, sem0, sem1):
    c = lax.axis_index("c")
    s = lax.axis_index("s")
    w = c * _NS + s
    _zero_slice(z_hbm, acc, s)
    plsc.subcore_barrier()

    def seg(m, carry):
        pltpu.sync_copy(src_hbm.at[w, m], src_v)
        pltpu.sync_copy(dst_hbm.at[w, m], dst_v)

        def pair(i, carry2):
            j0 = 2 * i
            j1 = j0 + 1
            cp0 = pltpu.async_copy(u_hbm.at[src_v.at[j0]], buf0, sem0)
            cp1 = pltpu.async_copy(u_hbm.at[src_v.at[j1]], buf1, sem1)
            cp0.wait()
            pltpu.sync_copy(buf0, acc.at[dst_v.at[j0]], add=True)
            cp1.wait()
            pltpu.sync_copy(buf1, acc.at[dst_v.at[j1]], add=True)
            return carry2

        lax.fori_loop(0, _PAIRS, pair, 0)
        cp = pltpu.async_copy(u_hbm.at[src_v.at[_CPS - 1]], buf0, sem0)
        cp.wait()
        pltpu.sync_copy(buf0, acc.at[dst_v.at[_CPS - 1]], add=True)
        return carry

    lax.fori_loop(0, _NSEG, seg, 0)
    plsc.subcore_barrier()
    _out_slice(acc, out_hbm, c, s)


@functools.cache
def _get_sc_agg():
    return pl.kernel(
        _sc_agg_body,
        out_type=jax.ShapeDtypeStruct((_NC, _N, _D), jnp.float32),
        scratch_types=[
            pltpu.VMEM((_CPS, _K), jnp.int32),
            pltpu.VMEM((_CPS, _K), jnp.int32),
            pltpu.VMEM((_K, _D), jnp.float32),
            pltpu.VMEM((_K, _D), jnp.float32),
            pltpu.VMEM_SHARED((_N, _D), jnp.float32),
            pltpu.SemaphoreType.DMA,
            pltpu.SemaphoreType.DMA,
        ],
        mesh=plsc.VectorSubcoreMesh(core_axis_name="c", subcore_axis_name="s"),
    )


def _sc_cnt_body(dst_hbm, ones_hbm, z_hbm, out_hbm, dst_v, ones_v, acc):
    c = lax.axis_index("c")
    s = lax.axis_index("s")
    w = c * _NS + s
    _zero_slice(z_hbm, acc, s)
    pltpu.sync_copy(dst_hbm.at[w], dst_v)
    pltpu.sync_copy(ones_hbm, ones_v)
    plsc.subcore_barrier()

    def one(j, carry):
        pltpu.sync_copy(ones_v, acc.at[dst_v.at[j]], add=True)
        return carry

    lax.fori_loop(0, _NCH, one, 0)
    plsc.subcore_barrier()
    _out_slice(acc, out_hbm, c, s)


@functools.cache
def _get_sc_cnt():
    return pl.kernel(
        _sc_cnt_body,
        out_type=jax.ShapeDtypeStruct((_NC, _N, _CW), jnp.float32),
        scratch_types=[
            pltpu.VMEM((_NCH, _K), jnp.int32),
            pltpu.VMEM((_K, _CW), jnp.float32),
            pltpu.VMEM_SHARED((_N, _CW), jnp.float32),
        ],
        mesh=plsc.VectorSubcoreMesh(core_axis_name="c", subcore_axis_name="s"),
    )


def _tc_init_body(x_ref, w_ref, b_ref, cnt_ref, hw_ref, u_ref, dinv_ref):
    deg = cnt_ref[0, :, 0:1] + cnt_ref[1, :, 0:1] + 1.0
    dinv = lax.rsqrt(deg)
    hw = jnp.dot(x_ref[...], w_ref[...], preferred_element_type=jnp.float32)
    hw = hw + b_ref[...]
    hw_ref[...] = hw
    u_ref[...] = hw * dinv
    dinv_ref[...] = dinv


_tc_init = pl.pallas_call(
    _tc_init_body,
    out_shape=(
        jax.ShapeDtypeStruct((_N, _D), jnp.float32),
        jax.ShapeDtypeStruct((_N, _D), jnp.float32),
        jax.ShapeDtypeStruct((_N, 1), jnp.float32),
    ),
)


def _bn_h(S_ref, hw_ref, dinv_ref, g_ref, be_ref, relu):
    dinv = dinv_ref[...]
    agg = (S_ref[0] + S_ref[1]) * dinv + hw_ref[...] * (dinv * dinv)
    mean = jnp.mean(agg, axis=0, keepdims=True)
    cen = agg - mean
    var = jnp.mean(cen * cen, axis=0, keepdims=True)
    h = cen / jnp.sqrt(var + 1e-5) * g_ref[...] + be_ref[...]
    if relu:
        h = jnp.maximum(h, 0.0)
    return h


def _tc_layer_body(S_ref, hw_ref, dinv_ref, g_ref, be_ref, wn_ref, bn_ref,
                   u_ref, hwo_ref):
    h = _bn_h(S_ref, hw_ref, dinv_ref, g_ref, be_ref, relu=True)
    hw2 = jnp.dot(h, wn_ref[...], preferred_element_type=jnp.float32)
    hw2 = hw2 + bn_ref[...]
    hwo_ref[...] = hw2
    u_ref[...] = hw2 * dinv_ref[...]


_tc_layer = pl.pallas_call(
    _tc_layer_body,
    out_shape=(
        jax.ShapeDtypeStruct((_N, _D), jnp.float32),
        jax.ShapeDtypeStruct((_N, _D), jnp.float32),
    ),
)


def _tc_final_body(S_ref, hw_ref, dinv_ref, g_ref, be_ref, bn1_ref, b1n_ref,
                   watt_ref, out_ref):
    h = _bn_h(S_ref, hw_ref, dinv_ref, g_ref, be_ref, relu=False)
    ohT = (b1n_ref[...] == lax.broadcasted_iota(jnp.int32, (_G, _N), 0))
    ohT = ohT.astype(jnp.float32)
    oh = (bn1_ref[...] == lax.broadcasted_iota(jnp.int32, (_N, _G), 1))
    oh = oh.astype(jnp.float32)
    gsum = jnp.dot(ohT, h, preferred_element_type=jnp.float32,
                   precision=lax.Precision.HIGHEST)
    counts = jnp.sum(ohT, axis=1, keepdims=True)
    gmean = gsum / jnp.maximum(counts, 1.0)
    ctx = jnp.tanh(jnp.dot(gmean, watt_ref[...],
                           preferred_element_type=jnp.float32))
    ctxb = jnp.dot(oh, ctx, preferred_element_type=jnp.float32,
                   precision=lax.Precision.HIGHEST)
    logit = jnp.sum(h * ctxb, axis=1, keepdims=True)
    scores = 1.0 / (1.0 + jnp.exp(-logit))
    out_ref[...] = jnp.dot(ohT, scores * h, preferred_element_type=jnp.float32,
                           precision=lax.Precision.HIGHEST)


_tc_final = pl.pallas_call(
    _tc_final_body,
    out_shape=jax.ShapeDtypeStruct((_G, _D), jnp.float32),
)


def kernel(x, edge_index, batch, Ws, bs, gammas, betas, W_att):
    src4 = edge_index[0].reshape(_NW, _NSEG, _CPS, _K)
    dst4 = edge_index[1].reshape(_NW, _NSEG, _CPS, _K)
    dst3 = edge_index[1].reshape(_NW, _NCH, _K)
    z128 = jnp.zeros((_RB, _D), jnp.float32)
    z16 = jnp.zeros((_RB, _CW), jnp.float32)
    ones16 = jnp.ones((_K, _CW), jnp.float32)
    bn1 = batch.reshape(_N, 1)
    b1n = batch.reshape(1, _N)

    sc_cnt = _get_sc_cnt()
    sc_agg = _get_sc_agg()
    cnt = sc_cnt(dst3, ones16, z16)
    hw, u, dinv = _tc_init(x, Ws[0], bs[0].reshape(1, _D), cnt)
    for l in range(_L - 1):
        S = sc_agg(u, src4, dst4, z128)
        u, hw = _tc_layer(S, hw, dinv, gammas[l].reshape(1, _D),
                          betas[l].reshape(1, _D), Ws[l + 1],
                          bs[l + 1].reshape(1, _D))
    S = sc_agg(u, src4, dst4, z128)
    out = _tc_final(S, hw, dinv, gammas[_L - 1].reshape(1, _D),
                    betas[_L - 1].reshape(1, _D), bn1, b1n, W_att)
    return out


# speedup vs baseline: 12.3675x; 1.1918x over previous
import functools

import jax
import jax.numpy as jnp
from jax import lax
from jax.experimental import pallas as pl
from jax.experimental.pallas import tpu as pltpu
from jax.experimental.pallas import tpu_sc as plsc

_N = 10000
_E = 320000
_D = 128
_L = 10
_G = 16

_NC = 2
_NS = 16
_NW = _NC * _NS
_K = 80
_EPW = _E // _NW
_NCH = _EPW // _K
_NSEG = 5
_CPS = _NCH // _NSEG
_TRIADS = (_CPS - 1) // 3
_RB = 632
_RL = _N - (_NS - 1) * _RB
_CW = _D

def _zero_slice(z_hbm, acc, s):
    @pl.when(s < _NS - 1)
    def _():
        pltpu.sync_copy(z_hbm, acc.at[pl.ds(s * _RB, _RB)])

    @pl.when(s == _NS - 1)
    def _():
        pltpu.sync_copy(z_hbm.at[pl.ds(0, _RL)],
                        acc.at[pl.ds((_NS - 1) * _RB, _RL)])


def _out_slice(acc, out_hbm, c, s):
    @pl.when(s < _NS - 1)
    def _():
        pltpu.sync_copy(acc.at[pl.ds(s * _RB, _RB)],
                        out_hbm.at[c, pl.ds(s * _RB, _RB)])

    @pl.when(s == _NS - 1)
    def _():
        pltpu.sync_copy(acc.at[pl.ds((_NS - 1) * _RB, _RL)],
                        out_hbm.at[c, pl.ds((_NS - 1) * _RB, _RL)])


def _sc_agg_body(u_hbm, src_hbm, dst_hbm, z_hbm, out_hbm,
                 src_v, dst_v, buf0, buf1, buf2,
                 acc, gs0, gs1, gs2, ss0, ss1, ss2):
    c = lax.axis_index("c")
    s = lax.axis_index("s")
    w = c * _NS + s
    _zero_slice(z_hbm, acc, s)
    plsc.subcore_barrier()

    bufs = (buf0, buf1, buf2)
    ssems = (ss0, ss1, ss2)
    gsems = (gs0, gs1, gs2)

    def wait_scatter(buf, sem, j):
        pltpu.make_async_copy(buf, acc.at[dst_v.at[j]], sem).wait()

    def seg(m, carry):
        pltpu.sync_copy(src_hbm.at[w, m], src_v)
        pltpu.sync_copy(dst_hbm.at[w, m], dst_v)

        def triad(q, carry2):
            j0 = 3 * q
            gs = []
            for t in range(3):
                @pl.when(q > 0)
                def _(t=t, j0=j0):
                    wait_scatter(bufs[t], ssems[t], j0 + t - 3)
                gs.append(pltpu.async_copy(
                    u_hbm.at[src_v.at[j0 + t]], bufs[t], gsems[t]))
            for t in range(3):
                gs[t].wait()
                pltpu.async_copy(bufs[t], acc.at[dst_v.at[j0 + t]], ssems[t],
                                 add=True)
            return carry2

        lax.fori_loop(0, _TRIADS, triad, 0)
        for t in range(3):
            wait_scatter(bufs[t], ssems[t], 3 * _TRIADS + t - 3)
        cp = pltpu.async_copy(u_hbm.at[src_v.at[_CPS - 1]], buf0, gs0)
        cp.wait()
        pltpu.sync_copy(buf0, acc.at[dst_v.at[_CPS - 1]], add=True)
        return carry

    lax.fori_loop(0, _NSEG, seg, 0)
    plsc.subcore_barrier()
    _out_slice(acc, out_hbm, c, s)


@functools.cache
def _get_sc_agg():
    return pl.kernel(
        _sc_agg_body,
        out_type=jax.ShapeDtypeStruct((_NC, _N, _D), jnp.float32),
        scratch_types=[
            pltpu.VMEM((_CPS, _K), jnp.int32),
            pltpu.VMEM((_CPS, _K), jnp.int32),
            pltpu.VMEM((_K, _D), jnp.float32),
            pltpu.VMEM((_K, _D), jnp.float32),
            pltpu.VMEM((_K, _D), jnp.float32),
            pltpu.VMEM_SHARED((_N, _D), jnp.float32),
            pltpu.SemaphoreType.DMA,
            pltpu.SemaphoreType.DMA,
            pltpu.SemaphoreType.DMA,
            pltpu.SemaphoreType.DMA,
            pltpu.SemaphoreType.DMA,
            pltpu.SemaphoreType.DMA,
        ],
        mesh=plsc.VectorSubcoreMesh(core_axis_name="c", subcore_axis_name="s"),
    )


def _sc_cnt_body(dst_hbm, ones_hbm, z_hbm, out_hbm, dst_v, ones_v, acc):
    c = lax.axis_index("c")
    s = lax.axis_index("s")
    w = c * _NS + s
    _zero_slice(z_hbm, acc, s)
    pltpu.sync_copy(dst_hbm.at[w], dst_v)
    pltpu.sync_copy(ones_hbm, ones_v)
    plsc.subcore_barrier()

    def one(j, carry):
        pltpu.sync_copy(ones_v, acc.at[dst_v.at[j]], add=True)
        return carry

    lax.fori_loop(0, _NCH, one, 0)
    plsc.subcore_barrier()
    _out_slice(acc, out_hbm, c, s)


@functools.cache
def _get_sc_cnt():
    return pl.kernel(
        _sc_cnt_body,
        out_type=jax.ShapeDtypeStruct((_NC, _N, _CW), jnp.float32),
        scratch_types=[
            pltpu.VMEM((_NCH, _K), jnp.int32),
            pltpu.VMEM((_K, _CW), jnp.float32),
            pltpu.VMEM_SHARED((_N, _CW), jnp.float32),
        ],
        mesh=plsc.VectorSubcoreMesh(core_axis_name="c", subcore_axis_name="s"),
    )


def _tc_init_body(x_ref, w_ref, b_ref, cnt_ref, hw_ref, u_ref, dinv_ref):
    deg = cnt_ref[0, :, 0:1] + cnt_ref[1, :, 0:1] + 1.0
    dinv = lax.rsqrt(deg)
    hw = jnp.dot(x_ref[...], w_ref[...], preferred_element_type=jnp.float32)
    hw = hw + b_ref[...]
    hw_ref[...] = hw
    u_ref[...] = hw * dinv
    dinv_ref[...] = dinv


_tc_init = pl.pallas_call(
    _tc_init_body,
    out_shape=(
        jax.ShapeDtypeStruct((_N, _D), jnp.float32),
        jax.ShapeDtypeStruct((_N, _D), jnp.float32),
        jax.ShapeDtypeStruct((_N, 1), jnp.float32),
    ),
)


def _bn_h(S_ref, hw_ref, dinv_ref, g_ref, be_ref, relu):
    dinv = dinv_ref[...]
    agg = (S_ref[0] + S_ref[1]) * dinv + hw_ref[...] * (dinv * dinv)
    mean = jnp.mean(agg, axis=0, keepdims=True)
    cen = agg - mean
    var = jnp.mean(cen * cen, axis=0, keepdims=True)
    h = cen / jnp.sqrt(var + 1e-5) * g_ref[...] + be_ref[...]
    if relu:
        h = jnp.maximum(h, 0.0)
    return h


def _tc_layer_body(S_ref, hw_ref, dinv_ref, g_ref, be_ref, wn_ref, bn_ref,
                   u_ref, hwo_ref):
    h = _bn_h(S_ref, hw_ref, dinv_ref, g_ref, be_ref, relu=True)
    hw2 = jnp.dot(h, wn_ref[...], preferred_element_type=jnp.float32)
    hw2 = hw2 + bn_ref[...]
    hwo_ref[...] = hw2
    u_ref[...] = hw2 * dinv_ref[...]


_tc_layer = pl.pallas_call(
    _tc_layer_body,
    out_shape=(
        jax.ShapeDtypeStruct((_N, _D), jnp.float32),
        jax.ShapeDtypeStruct((_N, _D), jnp.float32),
    ),
)


def _tc_final_body(S_ref, hw_ref, dinv_ref, g_ref, be_ref, bn1_ref, b1n_ref,
                   watt_ref, out_ref):
    h = _bn_h(S_ref, hw_ref, dinv_ref, g_ref, be_ref, relu=False)
    ohT = (b1n_ref[...] == lax.broadcasted_iota(jnp.int32, (_G, _N), 0))
    ohT = ohT.astype(jnp.float32)
    oh = (bn1_ref[...] == lax.broadcasted_iota(jnp.int32, (_N, _G), 1))
    oh = oh.astype(jnp.float32)
    gsum = jnp.dot(ohT, h, preferred_element_type=jnp.float32,
                   precision=lax.Precision.HIGHEST)
    counts = jnp.sum(ohT, axis=1, keepdims=True)
    gmean = gsum / jnp.maximum(counts, 1.0)
    ctx = jnp.tanh(jnp.dot(gmean, watt_ref[...],
                           preferred_element_type=jnp.float32))
    ctxb = jnp.dot(oh, ctx, preferred_element_type=jnp.float32,
                   precision=lax.Precision.HIGHEST)
    logit = jnp.sum(h * ctxb, axis=1, keepdims=True)
    scores = 1.0 / (1.0 + jnp.exp(-logit))
    out_ref[...] = jnp.dot(ohT, scores * h, preferred_element_type=jnp.float32,
                           precision=lax.Precision.HIGHEST)


_tc_final = pl.pallas_call(
    _tc_final_body,
    out_shape=jax.ShapeDtypeStruct((_G, _D), jnp.float32),
)


def kernel(x, edge_index, batch, Ws, bs, gammas, betas, W_att):
    src4 = edge_index[0].reshape(_NW, _NSEG, _CPS, _K)
    dst4 = edge_index[1].reshape(_NW, _NSEG, _CPS, _K)
    dst3 = edge_index[1].reshape(_NW, _NCH, _K)
    z128 = jnp.zeros((_RB, _D), jnp.float32)
    z16 = jnp.zeros((_RB, _CW), jnp.float32)
    ones16 = jnp.ones((_K, _CW), jnp.float32)
    bn1 = batch.reshape(_N, 1)
    b1n = batch.reshape(1, _N)

    sc_cnt = _get_sc_cnt()
    sc_agg = _get_sc_agg()
    cnt = sc_cnt(dst3, ones16, z16)
    hw, u, dinv = _tc_init(x, Ws[0], bs[0].reshape(1, _D), cnt)
    for l in range(_L - 1):
        S = sc_agg(u, src4, dst4, z128)
        u, hw = _tc_layer(S, hw, dinv, gammas[l].reshape(1, _D),
                          betas[l].reshape(1, _D), Ws[l + 1],
                          bs[l + 1].reshape(1, _D))
    S = sc_agg(u, src4, dst4, z128)
    out = _tc_final(S, hw, dinv, gammas[_L - 1].reshape(1, _D),
                    betas[_L - 1].reshape(1, _D), bn1, b1n, W_att)
    return out


# speedup vs baseline: 12.7002x; 1.0269x over previous
import functools

import jax
import jax.numpy as jnp
from jax import lax
from jax.experimental import pallas as pl
from jax.experimental.pallas import tpu as pltpu
from jax.experimental.pallas import tpu_sc as plsc

_N = 10000
_E = 320000
_D = 128
_L = 10
_G = 16

_NC = 2
_NS = 16
_NW = _NC * _NS
_K = 80
_EPW = _E // _NW
_NCH = _EPW // _K
_CPS = 24
_NSEG = 5
_TRIADS = _CPS // 3
_REM = _NCH - _NSEG * _CPS
_RB = 632
_RL = _N - (_NS - 1) * _RB
_CW = _D

def _zero_slice(z_hbm, acc, s):
    @pl.when(s < _NS - 1)
    def _():
        pltpu.sync_copy(z_hbm, acc.at[pl.ds(s * _RB, _RB)])

    @pl.when(s == _NS - 1)
    def _():
        pltpu.sync_copy(z_hbm.at[pl.ds(0, _RL)],
                        acc.at[pl.ds((_NS - 1) * _RB, _RL)])


def _out_slice(acc, out_hbm, c, s):
    @pl.when(s < _NS - 1)
    def _():
        pltpu.sync_copy(acc.at[pl.ds(s * _RB, _RB)],
                        out_hbm.at[c, pl.ds(s * _RB, _RB)])

    @pl.when(s == _NS - 1)
    def _():
        pltpu.sync_copy(acc.at[pl.ds((_NS - 1) * _RB, _RL)],
                        out_hbm.at[c, pl.ds((_NS - 1) * _RB, _RL)])


def _sc_agg_body(u_hbm, src_hbm, dst_hbm, z_hbm, out_hbm,
                 src_v, dst_v0, dst_v1, buf0, buf1, buf2,
                 acc, gs0, gs1, gs2, ss0, ss1, ss2):
    c = lax.axis_index("c")
    s = lax.axis_index("s")
    w = c * _NS + s
    _zero_slice(z_hbm, acc, s)
    plsc.subcore_barrier()

    bufs = (buf0, buf1, buf2)
    ssems = (ss0, ss1, ss2)
    gsems = (gs0, gs1, gs2)
    dslabs = (dst_v0, dst_v1)

    def wait_scatter(t, dst_v):
        pltpu.make_async_copy(bufs[t], acc.at[dst_v.at[0]], ssems[t]).wait()

    for m in range(_NSEG):
        dst_v = dslabs[m % 2]
        pltpu.sync_copy(src_hbm.at[w, pl.ds(m * _CPS, _CPS)], src_v)
        pltpu.sync_copy(dst_hbm.at[w, pl.ds(m * _CPS, _CPS)], dst_v)
        prev_dst = dslabs[(m - 1) % 2]

        def triad(q, carry2, m=m, dst_v=dst_v, prev_dst=prev_dst):
            j0 = 3 * q
            gs = []
            for t in range(3):
                if m == 0:
                    @pl.when(q > 0)
                    def _(t=t):
                        wait_scatter(t, dst_v)
                else:
                    wait_scatter(t, prev_dst)
                gs.append(pltpu.async_copy(
                    u_hbm.at[src_v.at[j0 + t]], bufs[t], gsems[t]))
            for t in range(3):
                gs[t].wait()
                pltpu.async_copy(bufs[t], acc.at[dst_v.at[j0 + t]], ssems[t],
                                 add=True)
            return carry2

        lax.fori_loop(0, _TRIADS, triad, 0)

    dst_v = dslabs[_NSEG % 2]
    pltpu.sync_copy(src_hbm.at[w, pl.ds(_NSEG * _CPS, _REM)],
                    src_v.at[pl.ds(0, _REM)])
    pltpu.sync_copy(dst_hbm.at[w, pl.ds(_NSEG * _CPS, _REM)],
                    dst_v.at[pl.ds(0, _REM)])
    gs = []
    for t in range(3):
        wait_scatter(t, dslabs[(_NSEG - 1) % 2])
        gs.append(pltpu.async_copy(u_hbm.at[src_v.at[t]], bufs[t], gsems[t]))
    for t in range(3):
        gs[t].wait()
        pltpu.async_copy(bufs[t], acc.at[dst_v.at[t]], ssems[t], add=True)
    for j in (3, 4):
        wait_scatter(j - 3, dst_v)
        cp = pltpu.async_copy(u_hbm.at[src_v.at[j]], bufs[j - 3], gsems[j - 3])
        cp.wait()
        pltpu.async_copy(bufs[j - 3], acc.at[dst_v.at[j]], ssems[j - 3],
                         add=True)
    for t in range(3):
        wait_scatter(t, dst_v)
    plsc.subcore_barrier()
    _out_slice(acc, out_hbm, c, s)


@functools.cache
def _get_sc_agg():
    return pl.kernel(
        _sc_agg_body,
        out_type=jax.ShapeDtypeStruct((_NC, _N, _D), jnp.float32),
        scratch_types=[
            pltpu.VMEM((_CPS, _K), jnp.int32),
            pltpu.VMEM((_CPS, _K), jnp.int32),
            pltpu.VMEM((_CPS, _K), jnp.int32),
            pltpu.VMEM((_K, _D), jnp.float32),
            pltpu.VMEM((_K, _D), jnp.float32),
            pltpu.VMEM((_K, _D), jnp.float32),
            pltpu.VMEM_SHARED((_N, _D), jnp.float32),
            pltpu.SemaphoreType.DMA,
            pltpu.SemaphoreType.DMA,
            pltpu.SemaphoreType.DMA,
            pltpu.SemaphoreType.DMA,
            pltpu.SemaphoreType.DMA,
            pltpu.SemaphoreType.DMA,
        ],
        mesh=plsc.VectorSubcoreMesh(core_axis_name="c", subcore_axis_name="s"),
    )


def _sc_cnt_body(dst_hbm, ones_hbm, z_hbm, out_hbm, dst_v, ones_v, acc, sem):
    c = lax.axis_index("c")
    s = lax.axis_index("s")
    w = c * _NS + s
    _zero_slice(z_hbm, acc, s)
    pltpu.sync_copy(dst_hbm.at[w], dst_v)
    pltpu.sync_copy(ones_hbm, ones_v)
    plsc.subcore_barrier()

    def wave(mw, carry):
        def fire(j, c2):
            pltpu.async_copy(ones_v, acc.at[dst_v.at[mw * 25 + j]], sem,
                             add=True)
            return c2

        lax.fori_loop(0, 25, fire, 0)

        def drain(j, c2):
            pltpu.make_async_copy(ones_v, acc.at[dst_v.at[0]], sem).wait()
            return c2

        lax.fori_loop(0, 25, drain, 0)
        return carry

    lax.fori_loop(0, _NCH // 25, wave, 0)
    plsc.subcore_barrier()
    _out_slice(acc, out_hbm, c, s)


@functools.cache
def _get_sc_cnt():
    return pl.kernel(
        _sc_cnt_body,
        out_type=jax.ShapeDtypeStruct((_NC, _N, _CW), jnp.float32),
        scratch_types=[
            pltpu.VMEM((_NCH, _K), jnp.int32),
            pltpu.VMEM((_K, _CW), jnp.float32),
            pltpu.VMEM_SHARED((_N, _CW), jnp.float32),
            pltpu.SemaphoreType.DMA,
        ],
        mesh=plsc.VectorSubcoreMesh(core_axis_name="c", subcore_axis_name="s"),
    )


def _tc_init_body(x_ref, w_ref, b_ref, cnt_ref, hw_ref, u_ref, dinv_ref):
    deg = cnt_ref[0, :, 0:1] + cnt_ref[1, :, 0:1] + 1.0
    dinv = lax.rsqrt(deg)
    hw = jnp.dot(x_ref[...], w_ref[...], preferred_element_type=jnp.float32)
    hw = hw + b_ref[...]
    hw_ref[...] = hw
    u_ref[...] = hw * dinv
    dinv_ref[...] = dinv


_tc_init = pl.pallas_call(
    _tc_init_body,
    out_shape=(
        jax.ShapeDtypeStruct((_N, _D), jnp.float32),
        jax.ShapeDtypeStruct((_N, _D), jnp.float32),
        jax.ShapeDtypeStruct((_N, 1), jnp.float32),
    ),
)


def _bn_h(S_ref, hw_ref, dinv_ref, g_ref, be_ref, relu):
    dinv = dinv_ref[...]
    agg = (S_ref[0] + S_ref[1]) * dinv + hw_ref[...] * (dinv * dinv)
    mean = jnp.mean(agg, axis=0, keepdims=True)
    cen = agg - mean
    var = jnp.mean(cen * cen, axis=0, keepdims=True)
    h = cen / jnp.sqrt(var + 1e-5) * g_ref[...] + be_ref[...]
    if relu:
        h = jnp.maximum(h, 0.0)
    return h


def _tc_layer_body(S_ref, hw_ref, dinv_ref, g_ref, be_ref, wn_ref, bn_ref,
                   u_ref, hwo_ref):
    h = _bn_h(S_ref, hw_ref, dinv_ref, g_ref, be_ref, relu=True)
    hw2 = jnp.dot(h, wn_ref[...], preferred_element_type=jnp.float32)
    hw2 = hw2 + bn_ref[...]
    hwo_ref[...] = hw2
    u_ref[...] = hw2 * dinv_ref[...]


_tc_layer = pl.pallas_call(
    _tc_layer_body,
    out_shape=(
        jax.ShapeDtypeStruct((_N, _D), jnp.float32),
        jax.ShapeDtypeStruct((_N, _D), jnp.float32),
    ),
)


def _tc_final_body(S_ref, hw_ref, dinv_ref, g_ref, be_ref, bn1_ref, b1n_ref,
                   watt_ref, out_ref):
    h = _bn_h(S_ref, hw_ref, dinv_ref, g_ref, be_ref, relu=False)
    ohT = (b1n_ref[...] == lax.broadcasted_iota(jnp.int32, (_G, _N), 0))
    ohT = ohT.astype(jnp.float32)
    oh = (bn1_ref[...] == lax.broadcasted_iota(jnp.int32, (_N, _G), 1))
    oh = oh.astype(jnp.float32)
    gsum = jnp.dot(ohT, h, preferred_element_type=jnp.float32,
                   precision=lax.Precision.HIGHEST)
    counts = jnp.sum(ohT, axis=1, keepdims=True)
    gmean = gsum / jnp.maximum(counts, 1.0)
    ctx = jnp.tanh(jnp.dot(gmean, watt_ref[...],
                           preferred_element_type=jnp.float32))
    ctxb = jnp.dot(oh, ctx, preferred_element_type=jnp.float32,
                   precision=lax.Precision.HIGHEST)
    logit = jnp.sum(h * ctxb, axis=1, keepdims=True)
    scores = 1.0 / (1.0 + jnp.exp(-logit))
    out_ref[...] = jnp.dot(ohT, scores * h, preferred_element_type=jnp.float32,
                           precision=lax.Precision.HIGHEST)


_tc_final = pl.pallas_call(
    _tc_final_body,
    out_shape=jax.ShapeDtypeStruct((_G, _D), jnp.float32),
)


def kernel(x, edge_index, batch, Ws, bs, gammas, betas, W_att):
    src3 = edge_index[0].reshape(_NW, _NCH, _K)
    dst3 = edge_index[1].reshape(_NW, _NCH, _K)
    z128 = jnp.zeros((_RB, _D), jnp.float32)
    z16 = jnp.zeros((_RB, _CW), jnp.float32)
    ones16 = jnp.ones((_K, _CW), jnp.float32)
    bn1 = batch.reshape(_N, 1)
    b1n = batch.reshape(1, _N)

    sc_cnt = _get_sc_cnt()
    sc_agg = _get_sc_agg()
    cnt = sc_cnt(dst3, ones16, z16)
    hw, u, dinv = _tc_init(x, Ws[0], bs[0].reshape(1, _D), cnt)
    for l in range(_L - 1):
        S = sc_agg(u, src3, dst3, z128)
        u, hw = _tc_layer(S, hw, dinv, gammas[l].reshape(1, _D),
                          betas[l].reshape(1, _D), Ws[l + 1],
                          bs[l + 1].reshape(1, _D))
    S = sc_agg(u, src3, dst3, z128)
    out = _tc_final(S, hw, dinv, gammas[_L - 1].reshape(1, _D),
                    betas[_L - 1].reshape(1, _D), bn1, b1n, W_att)
    return out
